# baseline (device time: 92968 ns/iter reference)
import jax
import jax.numpy as jnp
from jax import lax
from jax.experimental import pallas as pl
from jax.experimental.pallas import tpu as pltpu

N_DEV = 4
WINDOW = 128
HEAD_DIM = 128
N_HEADS_LOC = 8
SCALE = 0.08838834764831843

SQ = 1024
D = 1024
HALF = SQ // 2
QTR = SQ // 4
KWIN = QTR + 2 * WINDOW


def _dot(a, b, dims):
    return lax.dot_general(a, b, (dims, ((), ())),
                           preferred_element_type=jnp.float32)


def _compute_quarter(qidx, xb_ref, wqb_ref, kb_ref, vb_ref, wob_ref):
    off = qidx * QTR
    kstart = pl.multiple_of(jnp.clip(off - WINDOW, 0, SQ - KWIN), WINDOW)

    xh = xb_ref[pl.ds(off, QTR), :]
    q = jnp.dot(xh, wqb_ref[...], preferred_element_type=jnp.float32)
    q = q.astype(jnp.bfloat16)

    qi = lax.broadcasted_iota(jnp.int32, (QTR, KWIN), 0) + off
    ki = lax.broadcasted_iota(jnp.int32, (QTR, KWIN), 1) + kstart
    mask = jnp.abs(qi - ki) <= WINDOW

    acc = jnp.zeros((QTR, D), jnp.float32)
    for h in range(N_HEADS_LOC):
        cols = slice(h * HEAD_DIM, (h + 1) * HEAD_DIM)
        kh = kb_ref[pl.ds(kstart, KWIN), cols]
        vh = vb_ref[pl.ds(kstart, KWIN), cols]
        s = _dot(q[:, cols], kh, ((1,), (1,))) * SCALE
        w = jnp.exp(jnp.where(mask, s, jnp.float32(-1e9)))
        w = w / jnp.sum(w, axis=1, keepdims=True)
        ctx = _dot(w.astype(jnp.bfloat16), vh, ((1,), (0,)))
        acc = acc + _dot(ctx.astype(jnp.bfloat16),
                         wob_ref[pl.ds(h * HEAD_DIM, HEAD_DIM), :],
                         ((1,), (0,)))
    return acc


def _body(x_ref, wq_ref, k_ref, v_ref, wo_ref, out_ref,
          xb_ref, wqb_ref, wob_ref, kb_ref, vb_ref,
          sendA, recv_a, sbuf, recv_b, qsend, recv_b2, recv_a2,
          send_sems, recv_sems):
    my = lax.axis_index("i")
    pa = 3 - my
    pb = my ^ 1
    ha = my // 2
    my_h_off = ha * HALF
    ot_h_off = (1 - ha) * HALF
    my_q_off = my * QTR
    pb_q_off = pb * QTR
    qoff = my_q_off - my_h_off
    qoffp = QTR - qoff
    q_first = 2 * (1 - ha)

    barrier_sem = pltpu.get_barrier_semaphore()
    for nbr in (pa, pb):
        pl.semaphore_signal(
            barrier_sem, inc=1,
            device_id=(nbr,), device_id_type=pl.DeviceIdType.MESH,
        )

    xb_ref[...] = x_ref[...].astype(jnp.bfloat16)
    wqb_ref[...] = wq_ref[...].astype(jnp.bfloat16)
    wob_ref[...] = wo_ref[...].astype(jnp.bfloat16)
    kb_ref[...] = k_ref[...].astype(jnp.bfloat16)
    vb_ref[...] = v_ref[...].astype(jnp.bfloat16)

    def rc(src, dst, sem_idx, dev):
        return pltpu.make_async_remote_copy(
            src_ref=src, dst_ref=dst,
            send_sem=send_sems.at[sem_idx], recv_sem=recv_sems.at[sem_idx],
            device_id=(dev,), device_id_type=pl.DeviceIdType.MESH,
        )

    p = _compute_quarter(q_first, xb_ref, wqb_ref, kb_ref, vb_ref, wob_ref)
    sendA[pl.ds(0, QTR), :] = p.astype(jnp.bfloat16)
    pl.semaphore_wait(barrier_sem, 2)
    rdma_a1 = rc(sendA.at[pl.ds(0, QTR)], recv_a.at[pl.ds(0, QTR)], 0, pa)
    rdma_a1.start()

    p = _compute_quarter(q_first + 1, xb_ref, wqb_ref, kb_ref, vb_ref,
                         wob_ref)
    sendA[pl.ds(QTR, QTR), :] = p.astype(jnp.bfloat16)
    rdma_a2 = rc(sendA.at[pl.ds(QTR, QTR)], recv_a.at[pl.ds(QTR, QTR)], 1, pa)
    rdma_a2.start()

    p2b = _compute_quarter(pb, xb_ref, wqb_ref, kb_ref, vb_ref, wob_ref)
    rdma_a1.wait()
    rdma_a2.wait()
    sbuf[...] = (p2b + recv_a[pl.ds(qoffp, QTR), :].astype(jnp.float32)
                 ).astype(jnp.bfloat16)
    rdma_b = rc(sbuf, recv_b, 2, pb)
    rdma_b.start()

    p2a = _compute_quarter(my, xb_ref, wqb_ref, kb_ref, vb_ref, wob_ref)
    rdma_b.wait()
    myq = (p2a + recv_a[pl.ds(qoff, QTR), :].astype(jnp.float32)
           + recv_b[...].astype(jnp.float32))

    qsend[...] = myq.astype(jnp.bfloat16)
    rdma_b2 = rc(qsend, recv_b2, 3, pb)
    rdma_b2.start()
    rdma_a3a = rc(qsend, recv_a2.at[pl.ds(qoff, QTR)], 4, pa)
    rdma_a3a.start()
    out_ref[pl.ds(my_q_off, QTR), :] = myq
    rdma_b2.wait()
    rdma_a3b = rc(recv_b2, recv_a2.at[pl.ds(qoffp, QTR)], 5, pa)
    rdma_a3b.start()
    out_ref[pl.ds(pb_q_off, QTR), :] = recv_b2[...].astype(jnp.float32)
    rdma_a3a.wait()
    rdma_a3b.wait()
    out_ref[pl.ds(ot_h_off, HALF), :] = recv_a2[...].astype(jnp.float32)


def kernel(x, Wq, K_ext, V_ext, Wo):
    my = lax.axis_index("i")
    K = lax.dynamic_slice_in_dim(
        K_ext[0].reshape(SQ, 4 * D), my * D, D, axis=1)
    V = lax.dynamic_slice_in_dim(
        V_ext[0].reshape(SQ, 4 * D), my * D, D, axis=1)

    out = pl.pallas_call(
        _body,
        out_shape=jax.ShapeDtypeStruct((SQ, D), jnp.float32),
        in_specs=[pl.BlockSpec(memory_space=pltpu.VMEM)] * 5,
        out_specs=pl.BlockSpec(memory_space=pltpu.VMEM),
        scratch_shapes=[
            pltpu.VMEM((SQ, D), jnp.bfloat16),
            pltpu.VMEM((D, D), jnp.bfloat16),
            pltpu.VMEM((D, D), jnp.bfloat16),
            pltpu.VMEM((SQ, D), jnp.bfloat16),
            pltpu.VMEM((SQ, D), jnp.bfloat16),
            pltpu.VMEM((HALF, D), jnp.bfloat16),
            pltpu.VMEM((HALF, D), jnp.bfloat16),
            pltpu.VMEM((QTR, D), jnp.bfloat16),
            pltpu.VMEM((QTR, D), jnp.bfloat16),
            pltpu.VMEM((QTR, D), jnp.bfloat16),
            pltpu.VMEM((QTR, D), jnp.bfloat16),
            pltpu.VMEM((HALF, D), jnp.bfloat16),
            pltpu.SemaphoreType.DMA((6,)),
            pltpu.SemaphoreType.DMA((6,)),
        ],
        compiler_params=pltpu.CompilerParams(collective_id=0),
    )(x[0], Wq, K, V, Wo)
    return out[None]


# device time: 54330 ns/iter; 1.7112x vs baseline; 1.7112x over previous
import jax
import jax.numpy as jnp
from jax import lax
from jax.experimental import pallas as pl
from jax.experimental.pallas import tpu as pltpu

N_DEV = 4
WINDOW = 128
HEAD_DIM = 128
N_HEADS_LOC = 8
SCALE = 0.08838834764831843

SQ = 1024
D = 1024
HALF = SQ // 2
QTR = SQ // 4
KWIN = QTR + 2 * WINDOW


def _dot(a, b, dims):
    return lax.dot_general(a, b, (dims, ((), ())),
                           preferred_element_type=jnp.float32)


def _compute_quarter(qidx, xb_ref, wqb_ref, kb_ref, vb_ref, wob_ref):
    off = qidx * QTR
    kstart = pl.multiple_of(jnp.clip(off - WINDOW, 0, SQ - KWIN), WINDOW)

    xh = xb_ref[pl.ds(off, QTR), :]
    q = jnp.dot(xh, wqb_ref[...], preferred_element_type=jnp.float32)
    q = q.astype(jnp.bfloat16)

    qi = lax.broadcasted_iota(jnp.int32, (QTR, KWIN), 0) + off
    ki = lax.broadcasted_iota(jnp.int32, (QTR, KWIN), 1) + kstart
    mask = jnp.abs(qi - ki) <= WINDOW

    acc = jnp.zeros((QTR, D), jnp.float32)
    for h in range(N_HEADS_LOC):
        kh = kb_ref[h, pl.ds(kstart, KWIN), :]
        vh = vb_ref[h, pl.ds(kstart, KWIN), :]
        s = _dot(q[:, h * HEAD_DIM:(h + 1) * HEAD_DIM], kh,
                 ((1,), (1,))) * SCALE
        w = jnp.exp(jnp.where(mask, s, jnp.float32(-1e9)))
        w = w / jnp.sum(w, axis=1, keepdims=True)
        ctx = _dot(w.astype(jnp.bfloat16), vh, ((1,), (0,)))
        acc = acc + _dot(ctx.astype(jnp.bfloat16),
                         wob_ref[pl.ds(h * HEAD_DIM, HEAD_DIM), :],
                         ((1,), (0,)))
    return acc


def _body(x_ref, wq_ref, kb_ref, vb_ref, wo_ref, out_ref,
          xb_ref, wqb_ref, wob_ref,
          sendA, recv_a, sbuf, recv_b, qsend, recv_b2, recv_a2,
          send_sems, recv_sems):
    my = lax.axis_index("i")
    pa = 3 - my
    pb = my ^ 1
    ha = my // 2
    my_h_off = ha * HALF
    ot_h_off = (1 - ha) * HALF
    my_q_off = my * QTR
    pb_q_off = pb * QTR
    qoff = my_q_off - my_h_off
    qoffp = QTR - qoff
    q_first = 2 * (1 - ha)

    barrier_sem = pltpu.get_barrier_semaphore()
    for nbr in (pa, pb):
        pl.semaphore_signal(
            barrier_sem, inc=1,
            device_id=(nbr,), device_id_type=pl.DeviceIdType.MESH,
        )

    xb_ref[...] = x_ref[...].astype(jnp.bfloat16)
    wqb_ref[...] = wq_ref[...].astype(jnp.bfloat16)
    wob_ref[...] = wo_ref[...].astype(jnp.bfloat16)

    def rc(src, dst, sem_idx, dev):
        return pltpu.make_async_remote_copy(
            src_ref=src, dst_ref=dst,
            send_sem=send_sems.at[sem_idx], recv_sem=recv_sems.at[sem_idx],
            device_id=(dev,), device_id_type=pl.DeviceIdType.MESH,
        )

    p = _compute_quarter(q_first, xb_ref, wqb_ref, kb_ref, vb_ref, wob_ref)
    sendA[pl.ds(0, QTR), :] = p.astype(jnp.bfloat16)
    pl.semaphore_wait(barrier_sem, 2)
    rdma_a1 = rc(sendA.at[pl.ds(0, QTR)], recv_a.at[pl.ds(0, QTR)], 0, pa)
    rdma_a1.start()

    p = _compute_quarter(q_first + 1, xb_ref, wqb_ref, kb_ref, vb_ref,
                         wob_ref)
    sendA[pl.ds(QTR, QTR), :] = p.astype(jnp.bfloat16)
    rdma_a2 = rc(sendA.at[pl.ds(QTR, QTR)], recv_a.at[pl.ds(QTR, QTR)], 1, pa)
    rdma_a2.start()

    p2b = _compute_quarter(pb, xb_ref, wqb_ref, kb_ref, vb_ref, wob_ref)
    rdma_a1.wait()
    rdma_a2.wait()
    sbuf[...] = (p2b + recv_a[pl.ds(qoffp, QTR), :].astype(jnp.float32)
                 ).astype(jnp.bfloat16)
    rdma_b = rc(sbuf, recv_b, 2, pb)
    rdma_b.start()

    p2a = _compute_quarter(my, xb_ref, wqb_ref, kb_ref, vb_ref, wob_ref)
    rdma_b.wait()
    myq = (p2a + recv_a[pl.ds(qoff, QTR), :].astype(jnp.float32)
           + recv_b[...].astype(jnp.float32))

    qsend[...] = myq.astype(jnp.bfloat16)
    rdma_b2 = rc(qsend, recv_b2, 3, pb)
    rdma_b2.start()
    rdma_a3a = rc(qsend, recv_a2.at[pl.ds(qoff, QTR)], 4, pa)
    rdma_a3a.start()
    out_ref[pl.ds(my_q_off, QTR), :] = myq
    rdma_b2.wait()
    rdma_a3b = rc(recv_b2, recv_a2.at[pl.ds(qoffp, QTR)], 5, pa)
    rdma_a3b.start()
    out_ref[pl.ds(pb_q_off, QTR), :] = recv_b2[...].astype(jnp.float32)
    rdma_a3a.wait()
    rdma_a3b.wait()
    out_ref[pl.ds(ot_h_off, HALF), :] = recv_a2[...].astype(jnp.float32)


def kernel(x, Wq, K_ext, V_ext, Wo):
    my = lax.axis_index("i")
    K = lax.dynamic_slice_in_dim(K_ext[0], my * N_HEADS_LOC, N_HEADS_LOC,
                                 axis=1)
    V = lax.dynamic_slice_in_dim(V_ext[0], my * N_HEADS_LOC, N_HEADS_LOC,
                                 axis=1)
    K = jnp.transpose(K, (1, 0, 2)).astype(jnp.bfloat16)
    V = jnp.transpose(V, (1, 0, 2)).astype(jnp.bfloat16)

    out = pl.pallas_call(
        _body,
        out_shape=jax.ShapeDtypeStruct((SQ, D), jnp.float32),
        in_specs=[pl.BlockSpec(memory_space=pltpu.VMEM)] * 5,
        out_specs=pl.BlockSpec(memory_space=pltpu.VMEM),
        scratch_shapes=[
            pltpu.VMEM((SQ, D), jnp.bfloat16),
            pltpu.VMEM((D, D), jnp.bfloat16),
            pltpu.VMEM((D, D), jnp.bfloat16),
            pltpu.VMEM((HALF, D), jnp.bfloat16),
            pltpu.VMEM((HALF, D), jnp.bfloat16),
            pltpu.VMEM((QTR, D), jnp.bfloat16),
            pltpu.VMEM((QTR, D), jnp.bfloat16),
            pltpu.VMEM((QTR, D), jnp.bfloat16),
            pltpu.VMEM((QTR, D), jnp.bfloat16),
            pltpu.VMEM((HALF, D), jnp.bfloat16),
            pltpu.SemaphoreType.DMA((6,)),
            pltpu.SemaphoreType.DMA((6,)),
        ],
        compiler_params=pltpu.CompilerParams(collective_id=0),
    )(x[0], Wq, K, V, Wo)
    return out[None]


# device time: 53527 ns/iter; 1.7368x vs baseline; 1.0150x over previous
import jax
import jax.numpy as jnp
from jax import lax
from jax.experimental import pallas as pl
from jax.experimental.pallas import tpu as pltpu

N_DEV = 4
WINDOW = 128
HEAD_DIM = 128
N_HEADS_LOC = 8
SCALE = 0.08838834764831843

SQ = 1024
D = 1024
HALF = SQ // 2
QTR = SQ // 4
KWIN = QTR + 2 * WINDOW


def _dot(a, b, dims):
    return lax.dot_general(a, b, (dims, ((), ())),
                           preferred_element_type=jnp.float32)


def _compute_quarter(qidx, xb_ref, wqb_ref, kb_ref, vb_ref, wob_ref):
    off = qidx * QTR
    kstart = pl.multiple_of(jnp.clip(off - WINDOW, 0, SQ - KWIN), WINDOW)

    xh = xb_ref[pl.ds(off, QTR), :]
    q = jnp.dot(xh, wqb_ref[...], preferred_element_type=jnp.float32)
    q = q.astype(jnp.bfloat16)

    qi = lax.broadcasted_iota(jnp.int32, (QTR, KWIN), 0) + off
    ki = lax.broadcasted_iota(jnp.int32, (QTR, KWIN), 1) + kstart
    mask = jnp.abs(qi - ki) <= WINDOW

    acc = jnp.zeros((QTR, D), jnp.float32)
    for h in range(N_HEADS_LOC):
        kh = kb_ref[h, pl.ds(kstart, KWIN), :]
        vh = vb_ref[h, pl.ds(kstart, KWIN), :]
        s = _dot(q[:, h * HEAD_DIM:(h + 1) * HEAD_DIM], kh,
                 ((1,), (1,))) * SCALE
        w = jnp.exp(jnp.where(mask, s, jnp.float32(-1e9)))
        w = w / jnp.sum(w, axis=1, keepdims=True)
        ctx = _dot(w.astype(jnp.bfloat16), vh, ((1,), (0,)))
        acc = acc + _dot(ctx.astype(jnp.bfloat16),
                         wob_ref[pl.ds(h * HEAD_DIM, HEAD_DIM), :],
                         ((1,), (0,)))
    return acc


def _body(x_ref, wq_ref, kb_ref, vb_ref, wo_ref, out_ref,
          xb_ref, wqb_ref, wob_ref,
          sendA, recv_a, sbuf, recv_b, qsend, recv_b2, recv_a2,
          send_sems, recv_sems):
    my = lax.axis_index("i")
    pa = 3 - my
    pb = my ^ 1
    ha = my // 2
    my_h_off = ha * HALF
    ot_h_off = (1 - ha) * HALF
    my_q_off = my * QTR
    pb_q_off = pb * QTR
    qoff = my_q_off - my_h_off
    qoffp = QTR - qoff
    q_first = 2 * (1 - ha)

    barrier_sem = pltpu.get_barrier_semaphore()
    for nbr in (pa, pb):
        pl.semaphore_signal(
            barrier_sem, inc=1,
            device_id=(nbr,), device_id_type=pl.DeviceIdType.MESH,
        )

    xb_ref[...] = x_ref[...].astype(jnp.bfloat16)
    wqb_ref[...] = wq_ref[...].astype(jnp.bfloat16)
    wob_ref[...] = wo_ref[...].astype(jnp.bfloat16)

    def rc(src, dst, sem_idx, dev):
        return pltpu.make_async_remote_copy(
            src_ref=src, dst_ref=dst,
            send_sem=send_sems.at[sem_idx], recv_sem=recv_sems.at[sem_idx],
            device_id=(dev,), device_id_type=pl.DeviceIdType.MESH,
        )

    p = _compute_quarter(q_first, xb_ref, wqb_ref, kb_ref, vb_ref, wob_ref)
    sendA[pl.ds(0, QTR), :] = p.astype(jnp.bfloat16)
    pl.semaphore_wait(barrier_sem, 2)
    rdma_a1 = rc(sendA.at[pl.ds(0, QTR)], recv_a.at[pl.ds(0, QTR)], 0, pa)
    rdma_a1.start()

    p = _compute_quarter(q_first + 1, xb_ref, wqb_ref, kb_ref, vb_ref,
                         wob_ref)
    sendA[pl.ds(QTR, QTR), :] = p.astype(jnp.bfloat16)
    rdma_a2 = rc(sendA.at[pl.ds(QTR, QTR)], recv_a.at[pl.ds(QTR, QTR)], 1, pa)
    rdma_a2.start()

    p2b = _compute_quarter(pb, xb_ref, wqb_ref, kb_ref, vb_ref, wob_ref)
    rdma_a1.wait()
    rdma_a2.wait()
    sbuf[...] = (p2b + recv_a[pl.ds(qoffp, QTR), :].astype(jnp.float32)
                 ).astype(jnp.bfloat16)
    rdma_b = rc(sbuf, recv_b, 2, pb)
    rdma_b.start()

    p2a = _compute_quarter(my, xb_ref, wqb_ref, kb_ref, vb_ref, wob_ref)
    rdma_b.wait()
    myq = (p2a + recv_a[pl.ds(qoff, QTR), :].astype(jnp.float32)
           + recv_b[...].astype(jnp.float32))

    qsend[...] = myq.astype(jnp.bfloat16)
    rdma_b2 = rc(qsend, recv_b2, 3, pb)
    rdma_b2.start()
    rdma_a3a = rc(qsend, recv_a2.at[pl.ds(qoff, QTR)], 4, pa)
    rdma_a3a.start()
    out_ref[pl.ds(my_q_off, QTR), :] = qsend[...]
    rdma_b2.wait()
    rdma_a3b = rc(recv_b2, recv_a2.at[pl.ds(qoffp, QTR)], 5, pa)
    rdma_a3b.start()
    out_ref[pl.ds(pb_q_off, QTR), :] = recv_b2[...]
    rdma_a3a.wait()
    rdma_a3b.wait()
    out_ref[pl.ds(ot_h_off, HALF), :] = recv_a2[...]


def kernel(x, Wq, K_ext, V_ext, Wo):
    my = lax.axis_index("i")
    K = lax.dynamic_slice_in_dim(K_ext[0], my * N_HEADS_LOC, N_HEADS_LOC,
                                 axis=1)
    V = lax.dynamic_slice_in_dim(V_ext[0], my * N_HEADS_LOC, N_HEADS_LOC,
                                 axis=1)
    K = jnp.transpose(K, (1, 0, 2)).astype(jnp.bfloat16)
    V = jnp.transpose(V, (1, 0, 2)).astype(jnp.bfloat16)

    out = pl.pallas_call(
        _body,
        out_shape=jax.ShapeDtypeStruct((SQ, D), jnp.bfloat16),
        in_specs=[pl.BlockSpec(memory_space=pltpu.VMEM)] * 5,
        out_specs=pl.BlockSpec(memory_space=pltpu.VMEM),
        scratch_shapes=[
            pltpu.VMEM((SQ, D), jnp.bfloat16),
            pltpu.VMEM((D, D), jnp.bfloat16),
            pltpu.VMEM((D, D), jnp.bfloat16),
            pltpu.VMEM((HALF, D), jnp.bfloat16),
            pltpu.VMEM((HALF, D), jnp.bfloat16),
            pltpu.VMEM((QTR, D), jnp.bfloat16),
            pltpu.VMEM((QTR, D), jnp.bfloat16),
            pltpu.VMEM((QTR, D), jnp.bfloat16),
            pltpu.VMEM((QTR, D), jnp.bfloat16),
            pltpu.VMEM((HALF, D), jnp.bfloat16),
            pltpu.SemaphoreType.DMA((6,)),
            pltpu.SemaphoreType.DMA((6,)),
        ],
        compiler_params=pltpu.CompilerParams(collective_id=0),
    )(x[0], Wq, K, V, Wo)
    return out[None]
